# Initial kernel scaffold; baseline (speedup 1.0000x reference)
import jax
import jax.numpy as jnp
from jax.experimental import pallas as pl


def kernel(atom_features, pair_features, pair_split, atom_to_pair, W_aa, b_aa, W_pa, b_pa, W_ao, b_ao, W_ap, b_ap, W_pp, b_pp, W_po, b_po):
    raise NotImplementedError("write your pallas kernel here")


# speedup vs baseline: 3.5211x; 3.5211x over previous
import functools

import jax
import jax.numpy as jnp
from jax import lax
from jax.experimental import pallas as pl
from jax.experimental.pallas import tpu as pltpu
from jax.experimental.pallas import tpu_sc as plsc

N = 50000
E = 800000
D_A = 75
D_P = 14

CP = 64
XCOL = 128

NC = 2
NS = 16
NW = NC * NS

K = 128
EPT = 25088
EPAD = NW * EPT
NCHUNK = EPT // K

RNG = 12544
PA_ROWS = 4 * RNG
ACC_ROWS = 12800
OPT = RNG // NS


def _relu(x):
    return jnp.maximum(x, 0.0)


def _tc1_body(af_ref, wx_ref, out_ref):
    out_ref[...] = jnp.dot(af_ref[...], wx_ref[...],
                           preferred_element_type=jnp.float32)


def _tc2_body(pf_ref, wpa_ref, bpa_ref, out_ref):
    out_ref[...] = _relu(
        jnp.dot(pf_ref[...], wpa_ref[...],
                preferred_element_type=jnp.float32) + bpa_ref[...])


def _tc3_body(af_ref, pa_ref, waa_ref, baa_ref, wtop_ref, wbot_ref,
              bao_ref, out_ref):
    aa = _relu(jnp.dot(af_ref[...], waa_ref[...],
                       preferred_element_type=jnp.float32) + baa_ref[...])
    acc = jnp.dot(aa, wtop_ref[...], preferred_element_type=jnp.float32)
    acc = acc + jnp.dot(pa_ref[...], wbot_ref[...],
                        preferred_element_type=jnp.float32)
    out_ref[...] = _relu(acc + bao_ref[...])


def _tc4_body(s_ref, pf_ref, wtop_ref, wpp_ref, bpp_ref, wbot_ref,
              bpo_ref, out_ref):
    pp = _relu(jnp.dot(pf_ref[...], wpp_ref[...],
                       preferred_element_type=jnp.float32) + bpp_ref[...])
    acc = jnp.dot(s_ref[...], wtop_ref[...],
                  preferred_element_type=jnp.float32)
    acc = acc + jnp.dot(pp, wbot_ref[...], preferred_element_type=jnp.float32)
    out_ref[...] = _relu(acc + bpo_ref[...])


def _sc_seg_body(pae_hbm, split_hbm, pa_out, data_v, idx_v, acc_sh):
    cid = lax.axis_index("c")
    sid = lax.axis_index("s")
    wid = sid * NC + cid

    zeros16 = jnp.zeros((16,), jnp.float32)

    def _zrow(r, _):
        for cb in range(CP // 16):
            data_v[r, pl.ds(cb * 16, 16)] = zeros16
        return 0
    lax.fori_loop(0, K, _zrow, 0)

    rpt = ACC_ROWS // NS

    def _zero_acc():
        for z in range(rpt // K):
            pltpu.sync_copy(data_v, acc_sh.at[pl.ds(sid * rpt + z * K, K)])
        zrem = rpt % K
        pltpu.sync_copy(data_v.at[pl.ds(0, zrem)],
                        acc_sh.at[pl.ds(sid * rpt + (rpt // K) * K, zrem)])

    _zero_acc()
    plsc.subcore_barrier()

    for p in range(2):
        seg_base = (cid + 2 * p) * RNG

        def _seg_loop(g, _, seg_base=seg_base):
            start = wid * EPT + g * K
            pltpu.sync_copy(split_hbm.at[pl.ds(start, K)], idx_v)
            pltpu.sync_copy(pae_hbm.at[pl.ds(start, K)], data_v)

            def _fix(q, _):
                v = idx_v[pl.ds(q * 16, 16)]
                local = v - seg_base
                valid = (local >= 0) & (local < RNG)
                idx_v[pl.ds(q * 16, 16)] = jnp.where(valid, local, RNG)
                return 0
            lax.fori_loop(0, K // 16, _fix, 0)

            pltpu.sync_copy(data_v, acc_sh.at[idx_v], add=True)
            return 0

        lax.fori_loop(0, NCHUNK, _seg_loop, 0)
        plsc.subcore_barrier()

        for z in range(OPT // K):
            off = sid * OPT + z * K
            pltpu.sync_copy(acc_sh.at[pl.ds(off, K)], data_v)
            pltpu.sync_copy(data_v, pa_out.at[pl.ds(seg_base + off, K)])
        orem = OPT % K
        off = sid * OPT + (OPT // K) * K
        pltpu.sync_copy(acc_sh.at[pl.ds(off, orem)], data_v.at[pl.ds(0, orem)])
        pltpu.sync_copy(data_v.at[pl.ds(0, orem)],
                        pa_out.at[pl.ds(seg_base + off, orem)])

        if p == 0:
            plsc.subcore_barrier()
            lax.fori_loop(0, K, _zrow, 0)
            _zero_acc()
            plsc.subcore_barrier()


def _sc_gat_body(x12_hbm, bap_hbm, ii_hbm, jj_hbm, s_out,
                 ii_v, jj_v, gi_v, gj_v, s_v, bap_v, sem):
    cid = lax.axis_index("c")
    sid = lax.axis_index("s")
    wid = sid * NC + cid

    pltpu.sync_copy(bap_hbm, bap_v)

    def _gat_loop(g, _):
        start = wid * EPT + g * K
        pltpu.sync_copy(ii_hbm.at[pl.ds(start, K)], ii_v)
        pltpu.sync_copy(jj_hbm.at[pl.ds(start, K)], jj_v)
        ci = pltpu.async_copy(x12_hbm.at[ii_v], gi_v, sem)
        cj = pltpu.async_copy(x12_hbm.at[jj_v], gj_v, sem)
        ci.wait()
        cj.wait()

        def _row(r, _):
            for cb in range(CP // 16):
                o = cb * 16
                bb = bap_v[pl.ds(o, 16)]
                u = gi_v[r, pl.ds(o, 16)] + gj_v[r, pl.ds(64 + o, 16)] + bb
                v = gj_v[r, pl.ds(o, 16)] + gi_v[r, pl.ds(64 + o, 16)] + bb
                s_v[r, pl.ds(o, 16)] = (jnp.maximum(u, 0.0) +
                                        jnp.maximum(v, 0.0))
            return 0
        lax.fori_loop(0, K, _row, 0)

        pltpu.sync_copy(s_v, s_out.at[pl.ds(start, K)])
        return 0

    lax.fori_loop(0, NCHUNK, _gat_loop, 0)


def _sc_calls(pae, split_pad, x12, bap_pad, ii_pad, jj_pad):
    mesh = plsc.VectorSubcoreMesh(core_axis_name="c", subcore_axis_name="s")
    seg = functools.partial(
        pl.kernel,
        out_type=jax.ShapeDtypeStruct((PA_ROWS, CP), jnp.float32),
        mesh=mesh,
        scratch_types=[
            pltpu.VMEM((K, CP), jnp.float32),
            pltpu.VMEM((K,), jnp.int32),
            pltpu.VMEM_SHARED((ACC_ROWS, CP), jnp.float32),
        ],
    )(_sc_seg_body)
    pa = seg(pae, split_pad)

    gat = functools.partial(
        pl.kernel,
        out_type=jax.ShapeDtypeStruct((EPAD, CP), jnp.float32),
        mesh=mesh,
        scratch_types=[
            pltpu.VMEM((K,), jnp.int32),
            pltpu.VMEM((K,), jnp.int32),
            pltpu.VMEM((K, XCOL), jnp.float32),
            pltpu.VMEM((K, XCOL), jnp.float32),
            pltpu.VMEM((K, CP), jnp.float32),
            pltpu.VMEM((CP,), jnp.float32),
            pltpu.SemaphoreType.DMA,
        ],
    )(_sc_gat_body)
    s = gat(x12, bap_pad, ii_pad, jj_pad)
    return pa, s


@jax.jit
def _run(atom_features, pair_features, pair_split, atom_to_pair,
         W_aa, b_aa, W_pa, b_pa, W_ao, b_ao, W_ap, b_ap, W_pp, b_pp,
         W_po, b_po):
    f32 = jnp.float32

    W_x = jnp.zeros((D_A, XCOL), f32)
    W_x = W_x.at[:, 0:50].set(W_ap[:D_A])
    W_x = W_x.at[:, 64:114].set(W_ap[D_A:])

    W_pa_pad = jnp.zeros((D_P, CP), f32).at[:, :50].set(W_pa)
    b_pa_pad = jnp.zeros((1, CP), f32).at[0, :50].set(b_pa)
    b_ap_pad = jnp.zeros((CP,), f32).at[:50].set(b_ap)

    W_ao_top = W_ao[:100]
    W_ao_bot = jnp.zeros((CP, 50), f32).at[:50].set(W_ao[100:])
    W_po_top = jnp.zeros((CP, 50), f32).at[:50].set(W_po[:50])
    W_po_bot = W_po[50:]

    b_aa2 = b_aa.reshape(1, 100)
    b_ao2 = b_ao.reshape(1, 50)
    b_pp2 = b_pp.reshape(1, 50)
    b_po2 = b_po.reshape(1, 50)

    split_pad = jnp.concatenate(
        [pair_split.astype(jnp.int32), jnp.full((EPAD - E,), N, jnp.int32)])
    a2p = atom_to_pair.astype(jnp.int32)
    zpad = jnp.zeros((EPAD - E,), jnp.int32)
    ii_pad = jnp.concatenate([a2p[:, 0], zpad])
    jj_pad = jnp.concatenate([a2p[:, 1], zpad])

    BN = 2000
    x12 = pl.pallas_call(
        _tc1_body,
        grid=(N // BN,),
        in_specs=[
            pl.BlockSpec((BN, D_A), lambda i: (i, 0)),
            pl.BlockSpec((D_A, XCOL), lambda i: (0, 0)),
        ],
        out_specs=pl.BlockSpec((BN, XCOL), lambda i: (i, 0)),
        out_shape=jax.ShapeDtypeStruct((N, XCOL), f32),
    )(atom_features, W_x)

    BE = 4096
    GE = EPAD // BE
    pae = pl.pallas_call(
        _tc2_body,
        grid=(GE,),
        in_specs=[
            pl.BlockSpec((BE, D_P), lambda i: (i, 0)),
            pl.BlockSpec((D_P, CP), lambda i: (0, 0)),
            pl.BlockSpec((1, CP), lambda i: (0, 0)),
        ],
        out_specs=pl.BlockSpec((BE, CP), lambda i: (i, 0)),
        out_shape=jax.ShapeDtypeStruct((EPAD, CP), f32),
    )(pair_features, W_pa_pad, b_pa_pad)

    pa, s = _sc_calls(pae, split_pad, x12, b_ap_pad, ii_pad, jj_pad)

    A = pl.pallas_call(
        _tc3_body,
        grid=(N // BN,),
        in_specs=[
            pl.BlockSpec((BN, D_A), lambda i: (i, 0)),
            pl.BlockSpec((BN, CP), lambda i: (i, 0)),
            pl.BlockSpec((D_A, 100), lambda i: (0, 0)),
            pl.BlockSpec((1, 100), lambda i: (0, 0)),
            pl.BlockSpec((100, 50), lambda i: (0, 0)),
            pl.BlockSpec((CP, 50), lambda i: (0, 0)),
            pl.BlockSpec((1, 50), lambda i: (0, 0)),
        ],
        out_specs=pl.BlockSpec((BN, 50), lambda i: (i, 0)),
        out_shape=jax.ShapeDtypeStruct((N, 50), f32),
    )(atom_features, pa, W_aa, b_aa2, W_ao_top, W_ao_bot, b_ao2)

    P = pl.pallas_call(
        _tc4_body,
        grid=(GE,),
        in_specs=[
            pl.BlockSpec((BE, CP), lambda i: (i, 0)),
            pl.BlockSpec((BE, D_P), lambda i: (i, 0)),
            pl.BlockSpec((CP, 50), lambda i: (0, 0)),
            pl.BlockSpec((D_P, 50), lambda i: (0, 0)),
            pl.BlockSpec((1, 50), lambda i: (0, 0)),
            pl.BlockSpec((50, 50), lambda i: (0, 0)),
            pl.BlockSpec((1, 50), lambda i: (0, 0)),
        ],
        out_specs=pl.BlockSpec((BE, 50), lambda i: (i, 0)),
        out_shape=jax.ShapeDtypeStruct((E, 50), f32),
    )(s, pair_features, W_po_top, W_pp, b_pp2, W_po_bot, b_po2)

    return (A, P)


def kernel(atom_features, pair_features, pair_split, atom_to_pair,
           W_aa, b_aa, W_pa, b_pa, W_ao, b_ao, W_ap, b_ap, W_pp, b_pp,
           W_po, b_po):
    return _run(atom_features, pair_features, pair_split, atom_to_pair,
                W_aa, b_aa, W_pa, b_pa, W_ao, b_ao, W_ap, b_ap,
                W_pp, b_pp, W_po, b_po)


# speedup vs baseline: 3.9712x; 1.1278x over previous
import functools

import jax
import jax.numpy as jnp
from jax import lax
from jax.experimental import pallas as pl
from jax.experimental.pallas import tpu as pltpu
from jax.experimental.pallas import tpu_sc as plsc

N = 50000
E = 800000
D_A = 75
D_P = 14

CP = 64
XCOL = 128

NC = 2
NS = 16
NW = NC * NS

K = 128
EPT = 25088
EPAD = NW * EPT
NCHUNK = EPT // K

RNG = 12544
PA_ROWS = 4 * RNG
ACC_ROWS = 12800
OPT = RNG // NS


def _relu(x):
    return jnp.maximum(x, 0.0)


def _tc1_body(af_ref, wx_ref, out_ref):
    out_ref[...] = jnp.dot(af_ref[...], wx_ref[...],
                           preferred_element_type=jnp.float32)


def _tc2_body(pf_ref, wpa_ref, bpa_ref, out_ref):
    out_ref[...] = _relu(
        jnp.dot(pf_ref[...], wpa_ref[...],
                preferred_element_type=jnp.float32) + bpa_ref[...])


def _tc3_body(af_ref, pa_ref, waa_ref, baa_ref, wtop_ref, wbot_ref,
              bao_ref, out_ref):
    aa = _relu(jnp.dot(af_ref[...], waa_ref[...],
                       preferred_element_type=jnp.float32) + baa_ref[...])
    acc = jnp.dot(aa, wtop_ref[...], preferred_element_type=jnp.float32)
    acc = acc + jnp.dot(pa_ref[...], wbot_ref[...],
                        preferred_element_type=jnp.float32)
    out_ref[...] = _relu(acc + bao_ref[...])


def _tc4_body(s_ref, pf_ref, wtop_ref, wpp_ref, bpp_ref, wbot_ref,
              bpo_ref, out_ref):
    pp = _relu(jnp.dot(pf_ref[...], wpp_ref[...],
                       preferred_element_type=jnp.float32) + bpp_ref[...])
    acc = jnp.dot(s_ref[...], wtop_ref[...],
                  preferred_element_type=jnp.float32)
    acc = acc + jnp.dot(pp, wbot_ref[...], preferred_element_type=jnp.float32)
    out_ref[...] = _relu(acc + bpo_ref[...])


def _sc_seg_body(pae_hbm, split_hbm, pa_out, data_v, idx_v, acc_sh):
    cid = lax.axis_index("c")
    sid = lax.axis_index("s")
    wid = sid * NC + cid

    zeros16 = jnp.zeros((16,), jnp.float32)

    def _zrow(r, _):
        for cb in range(CP // 16):
            data_v[r, pl.ds(cb * 16, 16)] = zeros16
        return 0
    lax.fori_loop(0, K, _zrow, 0)

    rpt = ACC_ROWS // NS

    def _zero_acc():
        for z in range(rpt // K):
            pltpu.sync_copy(data_v, acc_sh.at[pl.ds(sid * rpt + z * K, K)])
        zrem = rpt % K
        pltpu.sync_copy(data_v.at[pl.ds(0, zrem)],
                        acc_sh.at[pl.ds(sid * rpt + (rpt // K) * K, zrem)])

    _zero_acc()
    plsc.subcore_barrier()

    for p in range(2):
        seg_base = (cid + 2 * p) * RNG

        def _seg_loop(g, _, seg_base=seg_base):
            start = wid * EPT + g * K
            pltpu.sync_copy(split_hbm.at[pl.ds(start, K)], idx_v)
            pltpu.sync_copy(pae_hbm.at[pl.ds(start, K)], data_v)

            def _fix(q, _):
                v = idx_v[pl.ds(q * 16, 16)]
                local = v - seg_base
                valid = (local >= 0) & (local < RNG)
                idx_v[pl.ds(q * 16, 16)] = jnp.where(valid, local, RNG)
                return 0
            lax.fori_loop(0, K // 16, _fix, 0)

            pltpu.sync_copy(data_v, acc_sh.at[idx_v], add=True)
            return 0

        lax.fori_loop(0, NCHUNK, _seg_loop, 0)
        plsc.subcore_barrier()

        for z in range(OPT // K):
            off = sid * OPT + z * K
            pltpu.sync_copy(acc_sh.at[pl.ds(off, K)], data_v)
            pltpu.sync_copy(data_v, pa_out.at[pl.ds(seg_base + off, K)])
        orem = OPT % K
        off = sid * OPT + (OPT // K) * K
        pltpu.sync_copy(acc_sh.at[pl.ds(off, orem)], data_v.at[pl.ds(0, orem)])
        pltpu.sync_copy(data_v.at[pl.ds(0, orem)],
                        pa_out.at[pl.ds(seg_base + off, orem)])

        if p == 0:
            plsc.subcore_barrier()
            lax.fori_loop(0, K, _zrow, 0)
            _zero_acc()
            plsc.subcore_barrier()


def _sc_gat_body(x12_hbm, bap_hbm, ii_hbm, jj_hbm, s_out,
                 ii_a, jj_a, gi_a, gj_a, sem_a,
                 ii_b, jj_b, gi_b, gj_b, sem_b,
                 s_v, bap_v):
    cid = lax.axis_index("c")
    sid = lax.axis_index("s")
    wid = sid * NC + cid
    base = wid * EPT

    pltpu.sync_copy(bap_hbm, bap_v)

    def _fire(g, ii_v, jj_v, gi_v, gj_v, sem):
        start = base + g * K
        pltpu.sync_copy(ii_hbm.at[pl.ds(start, K)], ii_v)
        pltpu.sync_copy(jj_hbm.at[pl.ds(start, K)], jj_v)
        pltpu.async_copy(x12_hbm.at[ii_v], gi_v, sem)
        pltpu.async_copy(x12_hbm.at[jj_v], gj_v, sem)

    def _step(g, ii_v, jj_v, gi_v, gj_v, sem,
              nii, njj, ngi, ngj, nsem):
        @pl.when(g + 1 < NCHUNK)
        def _():
            _fire(g + 1, nii, njj, ngi, ngj, nsem)

        pltpu.make_async_copy(x12_hbm.at[ii_v], gi_v, sem).wait()
        pltpu.make_async_copy(x12_hbm.at[jj_v], gj_v, sem).wait()

        def _row(r, _):
            for cb in range(CP // 16):
                o = cb * 16
                bb = bap_v[pl.ds(o, 16)]
                u = gi_v[r, pl.ds(o, 16)] + gj_v[r, pl.ds(64 + o, 16)] + bb
                v = gj_v[r, pl.ds(o, 16)] + gi_v[r, pl.ds(64 + o, 16)] + bb
                s_v[r, pl.ds(o, 16)] = (jnp.maximum(u, 0.0) +
                                        jnp.maximum(v, 0.0))
            return 0
        lax.fori_loop(0, K, _row, 0)

        pltpu.sync_copy(s_v, s_out.at[pl.ds(base + g * K, K)])

    _fire(0, ii_a, jj_a, gi_a, gj_a, sem_a)

    def _gat_loop(g, _):
        @pl.when(g % 2 == 0)
        def _():
            _step(g, ii_a, jj_a, gi_a, gj_a, sem_a,
                  ii_b, jj_b, gi_b, gj_b, sem_b)

        @pl.when(g % 2 == 1)
        def _():
            _step(g, ii_b, jj_b, gi_b, gj_b, sem_b,
                  ii_a, jj_a, gi_a, gj_a, sem_a)
        return 0

    lax.fori_loop(0, NCHUNK, _gat_loop, 0)


def _sc_calls(pae, split_pad, x12, bap_pad, ii_pad, jj_pad):
    mesh = plsc.VectorSubcoreMesh(core_axis_name="c", subcore_axis_name="s")
    seg = functools.partial(
        pl.kernel,
        out_type=jax.ShapeDtypeStruct((PA_ROWS, CP), jnp.float32),
        mesh=mesh,
        scratch_types=[
            pltpu.VMEM((K, CP), jnp.float32),
            pltpu.VMEM((K,), jnp.int32),
            pltpu.VMEM_SHARED((ACC_ROWS, CP), jnp.float32),
        ],
    )(_sc_seg_body)
    pa = seg(pae, split_pad)

    gat = functools.partial(
        pl.kernel,
        out_type=jax.ShapeDtypeStruct((EPAD, CP), jnp.float32),
        mesh=mesh,
        scratch_types=[
            pltpu.VMEM((K,), jnp.int32),
            pltpu.VMEM((K,), jnp.int32),
            pltpu.VMEM((K, XCOL), jnp.float32),
            pltpu.VMEM((K, XCOL), jnp.float32),
            pltpu.SemaphoreType.DMA,
            pltpu.VMEM((K,), jnp.int32),
            pltpu.VMEM((K,), jnp.int32),
            pltpu.VMEM((K, XCOL), jnp.float32),
            pltpu.VMEM((K, XCOL), jnp.float32),
            pltpu.SemaphoreType.DMA,
            pltpu.VMEM((K, CP), jnp.float32),
            pltpu.VMEM((CP,), jnp.float32),
        ],
    )(_sc_gat_body)
    s = gat(x12, bap_pad, ii_pad, jj_pad)
    return pa, s


@jax.jit
def _run(atom_features, pair_features, pair_split, atom_to_pair,
         W_aa, b_aa, W_pa, b_pa, W_ao, b_ao, W_ap, b_ap, W_pp, b_pp,
         W_po, b_po):
    f32 = jnp.float32

    W_x = jnp.zeros((D_A, XCOL), f32)
    W_x = W_x.at[:, 0:50].set(W_ap[:D_A])
    W_x = W_x.at[:, 64:114].set(W_ap[D_A:])

    W_pa_pad = jnp.zeros((D_P, CP), f32).at[:, :50].set(W_pa)
    b_pa_pad = jnp.zeros((1, CP), f32).at[0, :50].set(b_pa)
    b_ap_pad = jnp.zeros((CP,), f32).at[:50].set(b_ap)

    W_ao_top = W_ao[:100]
    W_ao_bot = jnp.zeros((CP, 50), f32).at[:50].set(W_ao[100:])
    W_po_top = jnp.zeros((CP, 50), f32).at[:50].set(W_po[:50])
    W_po_bot = W_po[50:]

    b_aa2 = b_aa.reshape(1, 100)
    b_ao2 = b_ao.reshape(1, 50)
    b_pp2 = b_pp.reshape(1, 50)
    b_po2 = b_po.reshape(1, 50)

    split_pad = jnp.concatenate(
        [pair_split.astype(jnp.int32), jnp.full((EPAD - E,), N, jnp.int32)])
    a2p = atom_to_pair.astype(jnp.int32)
    zpad = jnp.zeros((EPAD - E,), jnp.int32)
    ii_pad = jnp.concatenate([a2p[:, 0], zpad])
    jj_pad = jnp.concatenate([a2p[:, 1], zpad])

    BN = 2000
    x12 = pl.pallas_call(
        _tc1_body,
        grid=(N // BN,),
        in_specs=[
            pl.BlockSpec((BN, D_A), lambda i: (i, 0)),
            pl.BlockSpec((D_A, XCOL), lambda i: (0, 0)),
        ],
        out_specs=pl.BlockSpec((BN, XCOL), lambda i: (i, 0)),
        out_shape=jax.ShapeDtypeStruct((N, XCOL), f32),
    )(atom_features, W_x)

    BE = 4096
    GE = EPAD // BE
    pae = pl.pallas_call(
        _tc2_body,
        grid=(GE,),
        in_specs=[
            pl.BlockSpec((BE, D_P), lambda i: (i, 0)),
            pl.BlockSpec((D_P, CP), lambda i: (0, 0)),
            pl.BlockSpec((1, CP), lambda i: (0, 0)),
        ],
        out_specs=pl.BlockSpec((BE, CP), lambda i: (i, 0)),
        out_shape=jax.ShapeDtypeStruct((EPAD, CP), f32),
    )(pair_features, W_pa_pad, b_pa_pad)

    pa, s = _sc_calls(pae, split_pad, x12, b_ap_pad, ii_pad, jj_pad)

    A = pl.pallas_call(
        _tc3_body,
        grid=(N // BN,),
        in_specs=[
            pl.BlockSpec((BN, D_A), lambda i: (i, 0)),
            pl.BlockSpec((BN, CP), lambda i: (i, 0)),
            pl.BlockSpec((D_A, 100), lambda i: (0, 0)),
            pl.BlockSpec((1, 100), lambda i: (0, 0)),
            pl.BlockSpec((100, 50), lambda i: (0, 0)),
            pl.BlockSpec((CP, 50), lambda i: (0, 0)),
            pl.BlockSpec((1, 50), lambda i: (0, 0)),
        ],
        out_specs=pl.BlockSpec((BN, 50), lambda i: (i, 0)),
        out_shape=jax.ShapeDtypeStruct((N, 50), f32),
    )(atom_features, pa, W_aa, b_aa2, W_ao_top, W_ao_bot, b_ao2)

    P = pl.pallas_call(
        _tc4_body,
        grid=(GE,),
        in_specs=[
            pl.BlockSpec((BE, CP), lambda i: (i, 0)),
            pl.BlockSpec((BE, D_P), lambda i: (i, 0)),
            pl.BlockSpec((CP, 50), lambda i: (0, 0)),
            pl.BlockSpec((D_P, 50), lambda i: (0, 0)),
            pl.BlockSpec((1, 50), lambda i: (0, 0)),
            pl.BlockSpec((50, 50), lambda i: (0, 0)),
            pl.BlockSpec((1, 50), lambda i: (0, 0)),
        ],
        out_specs=pl.BlockSpec((BE, 50), lambda i: (i, 0)),
        out_shape=jax.ShapeDtypeStruct((E, 50), f32),
    )(s, pair_features, W_po_top, W_pp, b_pp2, W_po_bot, b_po2)

    return (A, P)


def kernel(atom_features, pair_features, pair_split, atom_to_pair,
           W_aa, b_aa, W_pa, b_pa, W_ao, b_ao, W_ap, b_ap, W_pp, b_pp,
           W_po, b_po):
    return _run(atom_features, pair_features, pair_split, atom_to_pair,
                W_aa, b_aa, W_pa, b_pa, W_ao, b_ao, W_ap, b_ap,
                W_pp, b_pp, W_po, b_po)
